# baseline (speedup 1.0000x reference)
import functools

import jax
import jax.numpy as jnp
from jax import lax
from jax.experimental import pallas as pl
from jax.experimental.pallas import tpu as pltpu
from jax.experimental.pallas import tpu_sc as plsc

N = 10000
E = 320000
R = 3
D = 128
DH = D // 2

NC = 2
NS = 16

K = 128
E_PAD = 327680
NBT = E_PAD // (NS * K)
NBH = NBT // NC
NB_REAL = E // K
BINS = 30720
DUMP_BIN = BINS - 1
N_PAD = 10240
ROWS_PER_TILE = N_PAD // NS


@functools.cache
def _get_mesh():
    return plsc.VectorSubcoreMesh(core_axis_name="c", subcore_axis_name="s",
                                  num_cores=NC, num_subcores=NS)


@functools.cache
def _get_sc_degscale():
    return functools.partial(
        pl.kernel,
        out_type=(
            jax.ShapeDtypeStruct((NC, NS, NBH, K), jnp.int32),
            jax.ShapeDtypeStruct((NC, NS, NBH, K), jnp.float32),
        ),
        mesh=_get_mesh(),
        scratch_types=[
            pltpu.VMEM((NBT, K), jnp.int32),
            pltpu.VMEM((NBT, K), jnp.int32),
            pltpu.VMEM((NBT, K), jnp.int32),
            pltpu.VMEM((K,), jnp.int32),
            pltpu.VMEM((NBH, K), jnp.int32),
            pltpu.VMEM((NBH, K), jnp.float32),
            pltpu.VMEM((K,), jnp.float32),
            pltpu.VMEM((BINS // NS,), jnp.float32),
            pltpu.VMEM((K,), jnp.float32),
            pltpu.VMEM_SHARED((BINS,), jnp.float32),
        ],
    )(_sc_degscale_body)


def _sc_degscale_body(dst_p, typ_p, src_p,
                      combo_out, s_out,
                      dstA, typA, srcA, keyb, riB, sB, degb, zb, ones, binsS):
    cid = lax.axis_index("c")
    sid = lax.axis_index("s")

    def fill(ref, n, val):
        def body(i, _):
            ref[pl.ds(i * 16, 16)] = jnp.full((16,), val, ref.dtype)
            return 0
        lax.fori_loop(0, n // 16, body, 0, unroll=False)

    fill(zb, BINS // NS, 0.0)
    fill(ones, K, 1.0)

    pltpu.sync_copy(zb, binsS.at[pl.ds(sid * (BINS // NS), BINS // NS)])
    plsc.subcore_barrier()

    pltpu.sync_copy(dst_p.at[sid], dstA)
    pltpu.sync_copy(typ_p.at[sid], typA)
    pltpu.sync_copy(src_p.at[sid], srcA)

    def hist_body(j, _):
        valid = (sid * NBT + j) < NB_REAL
        for q in range(K // 16):
            sl = pl.ds(q * 16, 16)
            kv = dstA[j, sl] * R + typA[j, sl]
            keyb[sl] = jnp.where(valid, kv, DUMP_BIN)
        pltpu.sync_copy(ones, binsS.at[keyb], add=True)
        return 0
    lax.fori_loop(0, NBT, hist_body, 0, unroll=False)
    plsc.subcore_barrier()

    def scale_body(j, _):
        jj = cid * NBH + j
        valid = (sid * NBT + jj) < NB_REAL
        for q in range(K // 16):
            sl = pl.ds(q * 16, 16)
            dd = dstA[jj, sl]
            tt = typA[jj, sl]
            keyb[sl] = dd * R + tt
            riB[j, sl] = ((tt * N + srcA[jj, sl]) << 14) | dd
        pltpu.sync_copy(binsS.at[keyb], degb)
        for q in range(K // 16):
            sl = pl.ds(q * 16, 16)
            sv = 1.0 / jnp.maximum(degb[sl], 1.0)
            sB[j, sl] = jnp.where(valid, sv, 0.0)
        return 0
    lax.fori_loop(0, NBH, scale_body, 0, unroll=False)

    pltpu.sync_copy(riB, combo_out.at[cid, sid])
    pltpu.sync_copy(sB, s_out.at[cid, sid])


CH = 40
_C0 = 3


@functools.cache
def _get_sc_aggregate():
    return functools.partial(
        pl.kernel,
        out_type=jax.ShapeDtypeStruct((NC, N_PAD, D), jnp.float32),
        mesh=_get_mesh(),
        scratch_types=[
            pltpu.VMEM((CH, K), jnp.int32),
            pltpu.VMEM((CH, K), jnp.int32),
            pltpu.VMEM((CH, K), jnp.float32),
            pltpu.VMEM((K, D // 2), jnp.int32),
            pltpu.VMEM((K, D // 2), jnp.int32),
            pltpu.VMEM((K, D), jnp.float32),
            pltpu.SemaphoreType.DMA,
            pltpu.SemaphoreType.DMA,
            pltpu.VMEM_SHARED((N_PAD, D), jnp.float32),
        ],
        compiler_params=pltpu.CompilerParams(use_tc_tiling_on_sc=False,
                                             needs_layout_passes=False),
    )(_sc_aggregate_body)


def _sc_aggregate_body(hw, combo4, s4, agg_out,
                       idxS, dstS, sS, rowsA, rowsB, scat, semA, semB, accS):
    cid = lax.axis_index("c")
    sid = lax.axis_index("s")

    def zrow(i, _):
        for d in range(D // 16):
            scat[i, pl.ds(d * 16, 16)] = jnp.zeros((16,), jnp.float32)
        return 0
    lax.fori_loop(0, K, zrow, 0, unroll=False)
    base = sid * ROWS_PER_TILE
    for t in range(ROWS_PER_TILE // K):
        pltpu.sync_copy(scat, accS.at[pl.ds(base + t * K, K)])
    plsc.subcore_barrier()

    def scale_scatter(buf, j):
        def qstep(q, _):
            sv16 = sS[j, pl.ds(q * 16, 16)]
            for l in range(16):
                i = q * 16 + l
                sv = sv16[l]
                for g in range(D // 32):
                    w = buf[i, pl.ds(g * 16, 16)]
                    f0 = plsc.bitcast(w << 16, jnp.float32)
                    f1 = plsc.bitcast(w & jnp.int32(-65536), jnp.float32)
                    scat[i, pl.ds(g * 32, 16)] = f0 * sv
                    scat[i, pl.ds(g * 32 + 16, 16)] = f1 * sv
            return 0
        lax.fori_loop(0, K // 16, qstep, 0, unroll=False)
        pltpu.sync_copy(scat, accS.at[dstS.at[j]], add=True)

    def chunk(c, _):
        ph = NBH // CH
        hc = c // ph
        off = pl.multiple_of((c % ph) * CH, 8)
        pltpu.sync_copy(combo4.at[hc, sid, pl.ds(off, CH)], idxS)
        pltpu.sync_copy(s4.at[hc, sid, pl.ds(off, CH)], sS)

        def unpack(j, _):
            for q in range(K // 16):
                sl = pl.ds(q * 16, 16)
                cc = idxS[j, sl]
                dstS[j, sl] = cc & 16383
                idxS[j, sl] = cc >> 14
            return 0
        lax.fori_loop(0, CH, unpack, 0, unroll=False)

        pltpu.async_copy(hw.at[idxS.at[0]], rowsA, semA)

        def pair(t, _):
            jA = 2 * t
            jB = 2 * t + 1
            pltpu.async_copy(hw.at[idxS.at[jB]], rowsB, semB)
            pltpu.make_async_copy(hw.at[idxS.at[jA]], rowsA, semA).wait()
            scale_scatter(rowsA, jA)

            @pl.when(t < CH // 2 - 1)
            def _():
                pltpu.async_copy(hw.at[idxS.at[jA + 2]], rowsA, semA)

            pltpu.make_async_copy(hw.at[idxS.at[jB]], rowsB, semB).wait()
            scale_scatter(rowsB, jB)
            return 0
        lax.fori_loop(0, CH // 2, pair, 0, unroll=False)
        return 0

    nchunks = NBH * NC // CH
    lo = jnp.where(cid == 0, 0, _C0)
    hi = jnp.where(cid == 0, _C0, nchunks)
    lax.fori_loop(lo, hi, chunk, 0, unroll=False)
    plsc.subcore_barrier()

    pltpu.sync_copy(accS.at[pl.ds(base, ROWS_PER_TILE)],
                    agg_out.at[cid, pl.ds(base, ROWS_PER_TILE)])


_BLK = 400
_NBLK = N // _BLK


def _tc_proj_body(x_ref, wp_ref, bp_ref, w_ref, out_ref):
    h = jnp.dot(x_ref[...], wp_ref[...],
                preferred_element_type=jnp.float32) + bp_ref[...]
    for r in range(R):
        hw = jnp.dot(h, w_ref[r], preferred_element_type=jnp.float32)
        out_ref[r] = hw.astype(jnp.bfloat16)


def _tc_proj(x, wp, bp, w):
    return pl.pallas_call(
        _tc_proj_body,
        grid=(_NBLK,),
        in_specs=[
            pl.BlockSpec((_BLK, D), lambda i: (i, 0)),
            pl.BlockSpec((D, D), lambda i: (0, 0)),
            pl.BlockSpec((1, D), lambda i: (0, 0)),
            pl.BlockSpec((R, D, D), lambda i: (0, 0, 0)),
        ],
        out_specs=pl.BlockSpec((R, _BLK, D), lambda i: (0, i, 0)),
        out_shape=jax.ShapeDtypeStruct((R, N, D), jnp.bfloat16),
    )(x, wp, bp, w)


def _tc_comb_body(agg_ref, p_ref, b_ref, w_ref, out_ref):
    s = agg_ref[0] + agg_ref[1]
    h = jax.nn.relu(jnp.dot(s, p_ref[...],
                            preferred_element_type=jnp.float32) + b_ref[...])
    for r in range(R):
        hw = jnp.dot(h, w_ref[r], preferred_element_type=jnp.float32)
        out_ref[r] = hw.astype(jnp.bfloat16)


def _tc_comb(agg, p, b, w):
    return pl.pallas_call(
        _tc_comb_body,
        grid=(_NBLK,),
        in_specs=[
            pl.BlockSpec((NC, _BLK, D), lambda i: (0, i, 0)),
            pl.BlockSpec((D, D), lambda i: (0, 0)),
            pl.BlockSpec((1, D), lambda i: (0, 0)),
            pl.BlockSpec((R, D, D), lambda i: (0, 0, 0)),
        ],
        out_specs=pl.BlockSpec((R, _BLK, D), lambda i: (0, i, 0)),
        out_shape=jax.ShapeDtypeStruct((R, N, D), jnp.bfloat16),
    )(agg, p, b, w)


def _tc_final_body(agg_ref, p_ref, b_ref, out_ref):
    s = agg_ref[0] + agg_ref[1]
    out_ref[...] = jax.nn.relu(jnp.dot(s, p_ref[...],
                               preferred_element_type=jnp.float32) + b_ref[...])


def _tc_final(agg, p, b):
    return pl.pallas_call(
        _tc_final_body,
        grid=(_NBLK,),
        in_specs=[
            pl.BlockSpec((NC, _BLK, D), lambda i: (0, i, 0)),
            pl.BlockSpec((D, D), lambda i: (0, 0)),
            pl.BlockSpec((1, D), lambda i: (0, 0)),
        ],
        out_specs=pl.BlockSpec((_BLK, D), lambda i: (i, 0)),
        out_shape=jax.ShapeDtypeStruct((N, D), jnp.float32),
    )(agg, p, b)


def _unperm_matrix():
    import numpy as np
    perm = np.zeros((D, D), dtype=np.float32)
    for e in range(D):
        g, r2 = divmod(e, 32)
        j, b = divmod(r2, 2)
        perm[32 * g + 16 * b + j, e] = 1.0
    return jnp.asarray(perm)


def kernel(x, edge_index, edge_type, W_proj, b_proj, W1, b1, W2, b2):
    pad = E_PAD - E
    src = jnp.concatenate([edge_index[0], jnp.zeros((pad,), jnp.int32)])
    dst = jnp.concatenate([edge_index[1], jnp.zeros((pad,), jnp.int32)])
    et = jnp.concatenate([edge_type, jnp.zeros((pad,), jnp.int32)])

    dst_p = dst.reshape(NS, NBT, K)
    typ_p = et.reshape(NS, NBT, K)
    src_p = src.reshape(NS, NBT, K)

    sc_degscale = _get_sc_degscale()
    sc_aggregate = _get_sc_aggregate()
    combo4, s4 = sc_degscale(dst_p, typ_p, src_p)

    p_mat = _unperm_matrix()
    hw1 = _tc_proj(x, W_proj, b_proj.reshape(1, D), W1)
    hw1_i = lax.bitcast_convert_type(
        hw1.reshape(R * N, D // 2, 2), jnp.int32)
    agg1 = sc_aggregate(hw1_i, combo4, s4)
    hw2 = _tc_comb(agg1, p_mat, b1.reshape(1, D), W2)
    hw2_i = lax.bitcast_convert_type(
        hw2.reshape(R * N, D // 2, 2), jnp.int32)
    agg2 = sc_aggregate(hw2_i, combo4, s4)
    return _tc_final(agg2, p_mat, b2.reshape(1, D))


# speedup vs baseline: 1.5586x; 1.5586x over previous
import functools

import jax
import jax.numpy as jnp
from jax import lax
from jax.experimental import pallas as pl
from jax.experimental.pallas import tpu as pltpu
from jax.experimental.pallas import tpu_sc as plsc

N = 10000
E = 320000
R = 3
D = 128
DH = D // 2

NC = 2
NS = 16

K = 128
E_PAD = 327680
NBT = E_PAD // (NS * K)
NBH = NBT // NC
NB_REAL = E // K
BINS = 30720
DUMP_BIN = BINS - 1
N_PAD = 10240
ROWS_PER_TILE = N_PAD // NS


@functools.cache
def _get_mesh():
    return plsc.VectorSubcoreMesh(core_axis_name="c", subcore_axis_name="s",
                                  num_cores=NC, num_subcores=NS)


@functools.cache
def _get_sc_degscale():
    return functools.partial(
        pl.kernel,
        out_type=(
            jax.ShapeDtypeStruct((NC, NS, NBH, K), jnp.int32),
            jax.ShapeDtypeStruct((NC, NS, NBH, K), jnp.float32),
        ),
        mesh=_get_mesh(),
        scratch_types=[
            pltpu.VMEM((NBT, K), jnp.int32),
            pltpu.VMEM((NBT, K), jnp.int32),
            pltpu.VMEM((NBT, K), jnp.int32),
            pltpu.VMEM((K,), jnp.int32),
            pltpu.VMEM((NBH, K), jnp.int32),
            pltpu.VMEM((NBH, K), jnp.float32),
            pltpu.VMEM((K,), jnp.float32),
            pltpu.VMEM((BINS // NS,), jnp.float32),
            pltpu.VMEM((K,), jnp.float32),
            pltpu.VMEM_SHARED((BINS,), jnp.float32),
        ],
    )(_sc_degscale_body)


def _sc_degscale_body(dst_p, typ_p, src_p,
                      combo_out, s_out,
                      dstA, typA, srcA, keyb, riB, sB, degb, zb, ones, binsS):
    cid = lax.axis_index("c")
    sid = lax.axis_index("s")

    def fill(ref, n, val):
        def body(i, _):
            ref[pl.ds(i * 16, 16)] = jnp.full((16,), val, ref.dtype)
            return 0
        lax.fori_loop(0, n // 16, body, 0, unroll=False)

    fill(zb, BINS // NS, 0.0)
    fill(ones, K, 1.0)

    pltpu.sync_copy(zb, binsS.at[pl.ds(sid * (BINS // NS), BINS // NS)])
    plsc.subcore_barrier()

    pltpu.sync_copy(dst_p.at[sid], dstA)
    pltpu.sync_copy(typ_p.at[sid], typA)
    pltpu.sync_copy(src_p.at[sid], srcA)

    def hist_body(j, _):
        valid = (sid * NBT + j) < NB_REAL
        for q in range(K // 16):
            sl = pl.ds(q * 16, 16)
            kv = dstA[j, sl] * R + typA[j, sl]
            keyb[sl] = jnp.where(valid, kv, DUMP_BIN)
        pltpu.sync_copy(ones, binsS.at[keyb], add=True)
        return 0
    lax.fori_loop(0, NBT, hist_body, 0, unroll=False)
    plsc.subcore_barrier()

    def scale_body(j, _):
        jj = cid * NBH + j
        valid = (sid * NBT + jj) < NB_REAL
        for q in range(K // 16):
            sl = pl.ds(q * 16, 16)
            dd = dstA[jj, sl]
            tt = typA[jj, sl]
            keyb[sl] = dd * R + tt
            riB[j, sl] = ((tt * N + srcA[jj, sl]) << 14) | dd
        pltpu.sync_copy(binsS.at[keyb], degb)
        for q in range(K // 16):
            sl = pl.ds(q * 16, 16)
            sv = 1.0 / jnp.maximum(degb[sl], 1.0)
            sB[j, sl] = jnp.where(valid, sv, 0.0)
        return 0
    lax.fori_loop(0, NBH, scale_body, 0, unroll=False)

    pltpu.sync_copy(riB, combo_out.at[cid, sid])
    pltpu.sync_copy(sB, s_out.at[cid, sid])


CH = 40
_C0 = 3


@functools.cache
def _get_sc_aggregate():
    return functools.partial(
        pl.kernel,
        out_type=jax.ShapeDtypeStruct((NC, N_PAD, D), jnp.float32),
        mesh=_get_mesh(),
        scratch_types=[
            pltpu.VMEM((CH, K), jnp.int32),
            pltpu.VMEM((CH, K), jnp.int32),
            pltpu.VMEM((CH, K), jnp.float32),
            pltpu.VMEM((K, D), jnp.float32),
            pltpu.VMEM((K, D), jnp.float32),
            pltpu.SemaphoreType.DMA,
            pltpu.SemaphoreType.DMA,
            pltpu.VMEM_SHARED((N_PAD, D), jnp.float32),
        ],
    )(_sc_aggregate_body)


def _sc_aggregate_body(hw, combo4, s4, agg_out,
                       idxS, dstS, sS, rowsA, rowsB, semA, semB, accS):
    cid = lax.axis_index("c")
    sid = lax.axis_index("s")

    def zrow(i, _):
        for d in range(D // 16):
            rowsA[i, pl.ds(d * 16, 16)] = jnp.zeros((16,), jnp.float32)
        return 0
    lax.fori_loop(0, K, zrow, 0, unroll=False)
    base = sid * ROWS_PER_TILE
    for t in range(ROWS_PER_TILE // K):
        pltpu.sync_copy(rowsA, accS.at[pl.ds(base + t * K, K)])
    plsc.subcore_barrier()

    def scale_scatter(buf, j):
        def qstep(q, _):
            sv16 = sS[j, pl.ds(q * 16, 16)]
            for l in range(16):
                i = q * 16 + l
                sv = sv16[l]
                for d in range(D // 16):
                    sl = pl.ds(d * 16, 16)
                    buf[i, sl] = buf[i, sl] * sv
            return 0
        lax.fori_loop(0, K // 16, qstep, 0, unroll=False)
        pltpu.sync_copy(buf, accS.at[dstS.at[j]], add=True)

    def chunk(c, _):
        ph = NBH // CH
        hc = c // ph
        off = pl.multiple_of((c % ph) * CH, 8)
        pltpu.sync_copy(combo4.at[hc, sid, pl.ds(off, CH)], idxS)
        pltpu.sync_copy(s4.at[hc, sid, pl.ds(off, CH)], sS)

        def unpack(j, _):
            for q in range(K // 16):
                sl = pl.ds(q * 16, 16)
                cc = idxS[j, sl]
                dstS[j, sl] = cc & 16383
                idxS[j, sl] = cc >> 14
            return 0
        lax.fori_loop(0, CH, unpack, 0, unroll=False)

        pltpu.async_copy(hw.at[idxS.at[0]], rowsA, semA)

        def pair(t, _):
            jA = 2 * t
            jB = 2 * t + 1
            pltpu.async_copy(hw.at[idxS.at[jB]], rowsB, semB)
            pltpu.make_async_copy(hw.at[idxS.at[jA]], rowsA, semA).wait()
            scale_scatter(rowsA, jA)

            @pl.when(t < CH // 2 - 1)
            def _():
                pltpu.async_copy(hw.at[idxS.at[jA + 2]], rowsA, semA)

            pltpu.make_async_copy(hw.at[idxS.at[jB]], rowsB, semB).wait()
            scale_scatter(rowsB, jB)
            return 0
        lax.fori_loop(0, CH // 2, pair, 0, unroll=False)
        return 0

    nchunks = NBH * NC // CH
    lo = jnp.where(cid == 0, 0, _C0)
    hi = jnp.where(cid == 0, _C0, nchunks)
    lax.fori_loop(lo, hi, chunk, 0, unroll=False)
    plsc.subcore_barrier()

    pltpu.sync_copy(accS.at[pl.ds(base, ROWS_PER_TILE)],
                    agg_out.at[cid, pl.ds(base, ROWS_PER_TILE)])


_BLK = 400
_NBLK = N // _BLK


def _tc_proj_body(x_ref, wp_ref, bp_ref, w_ref, out_ref):
    h = jnp.dot(x_ref[...], wp_ref[...],
                preferred_element_type=jnp.float32) + bp_ref[...]
    for r in range(R):
        out_ref[r] = jnp.dot(h, w_ref[r], preferred_element_type=jnp.float32)


def _tc_proj(x, wp, bp, w):
    return pl.pallas_call(
        _tc_proj_body,
        grid=(_NBLK,),
        in_specs=[
            pl.BlockSpec((_BLK, D), lambda i: (i, 0)),
            pl.BlockSpec((D, D), lambda i: (0, 0)),
            pl.BlockSpec((1, D), lambda i: (0, 0)),
            pl.BlockSpec((R, D, D), lambda i: (0, 0, 0)),
        ],
        out_specs=pl.BlockSpec((R, _BLK, D), lambda i: (0, i, 0)),
        out_shape=jax.ShapeDtypeStruct((R, N, D), jnp.float32),
    )(x, wp, bp, w)


def _tc_comb_body(agg_ref, b_ref, w_ref, out_ref):
    h = jax.nn.relu(agg_ref[0] + agg_ref[1] + b_ref[...])
    for r in range(R):
        out_ref[r] = jnp.dot(h, w_ref[r], preferred_element_type=jnp.float32)


def _tc_comb(agg, b, w):
    return pl.pallas_call(
        _tc_comb_body,
        grid=(_NBLK,),
        in_specs=[
            pl.BlockSpec((NC, _BLK, D), lambda i: (0, i, 0)),
            pl.BlockSpec((1, D), lambda i: (0, 0)),
            pl.BlockSpec((R, D, D), lambda i: (0, 0, 0)),
        ],
        out_specs=pl.BlockSpec((R, _BLK, D), lambda i: (0, i, 0)),
        out_shape=jax.ShapeDtypeStruct((R, N, D), jnp.float32),
    )(agg, b, w)


def _tc_final_body(agg_ref, b_ref, out_ref):
    out_ref[...] = jax.nn.relu(agg_ref[0] + agg_ref[1] + b_ref[...])


def _tc_final(agg, b):
    return pl.pallas_call(
        _tc_final_body,
        grid=(_NBLK,),
        in_specs=[
            pl.BlockSpec((NC, _BLK, D), lambda i: (0, i, 0)),
            pl.BlockSpec((1, D), lambda i: (0, 0)),
        ],
        out_specs=pl.BlockSpec((_BLK, D), lambda i: (i, 0)),
        out_shape=jax.ShapeDtypeStruct((N, D), jnp.float32),
    )(agg, b)


def kernel(x, edge_index, edge_type, W_proj, b_proj, W1, b1, W2, b2):
    pad = E_PAD - E
    src = jnp.concatenate([edge_index[0], jnp.zeros((pad,), jnp.int32)])
    dst = jnp.concatenate([edge_index[1], jnp.zeros((pad,), jnp.int32)])
    et = jnp.concatenate([edge_type, jnp.zeros((pad,), jnp.int32)])

    dst_p = dst.reshape(NS, NBT, K)
    typ_p = et.reshape(NS, NBT, K)
    src_p = src.reshape(NS, NBT, K)

    sc_degscale = _get_sc_degscale()
    sc_aggregate = _get_sc_aggregate()
    combo4, s4 = sc_degscale(dst_p, typ_p, src_p)

    hw1 = _tc_proj(x, W_proj, b_proj.reshape(1, D), W1)
    agg1 = sc_aggregate(hw1.reshape(R * N, D), combo4, s4)
    hw2 = _tc_comb(agg1, b1.reshape(1, D), W2)
    agg2 = sc_aggregate(hw2.reshape(R * N, D), combo4, s4)
    return _tc_final(agg2, b2.reshape(1, D))


# speedup vs baseline: 1.5844x; 1.0166x over previous
import functools

import jax
import jax.numpy as jnp
from jax import lax
from jax.experimental import pallas as pl
from jax.experimental.pallas import tpu as pltpu
from jax.experimental.pallas import tpu_sc as plsc

N = 10000
E = 320000
R = 3
D = 128
DH = D // 2

NC = 2
NS = 16

K = 128
E_PAD = 327680
NBT = E_PAD // (NS * K)
NBH = NBT // NC
NB_REAL = E // K
BINS = 30720
DUMP_BIN = BINS - 1
N_PAD = 10240
ROWS_PER_TILE = N_PAD // NS


@functools.cache
def _get_mesh():
    return plsc.VectorSubcoreMesh(core_axis_name="c", subcore_axis_name="s",
                                  num_cores=NC, num_subcores=NS)


@functools.cache
def _get_sc_degscale():
    return functools.partial(
        pl.kernel,
        out_type=(
            jax.ShapeDtypeStruct((NC, NS, NBH, K), jnp.int32),
            jax.ShapeDtypeStruct((NC, NS, NBH, K), jnp.float32),
        ),
        mesh=_get_mesh(),
        scratch_types=[
            pltpu.VMEM((NBT, K), jnp.int32),
            pltpu.VMEM((NBT, K), jnp.int32),
            pltpu.VMEM((NBT, K), jnp.int32),
            pltpu.VMEM((K,), jnp.int32),
            pltpu.VMEM((K,), jnp.int32),
            pltpu.VMEM((NBH, K), jnp.int32),
            pltpu.VMEM((NBH, K), jnp.float32),
            pltpu.VMEM((K,), jnp.float32),
            pltpu.VMEM((K,), jnp.float32),
            pltpu.VMEM((BINS // NS,), jnp.float32),
            pltpu.VMEM((K,), jnp.float32),
            pltpu.SemaphoreType.DMA,
            pltpu.SemaphoreType.DMA,
            pltpu.VMEM_SHARED((BINS,), jnp.float32),
        ],
    )(_sc_degscale_body)


def _sc_degscale_body(dst_p, typ_p, src_p,
                      combo_out, s_out,
                      dstA, typA, srcA, keyb, keyb2, riB, sB, degb, degb2,
                      zb, ones, hsemA, hsemB, binsS):
    cid = lax.axis_index("c")
    sid = lax.axis_index("s")

    def fill(ref, n, val):
        def body(i, _):
            ref[pl.ds(i * 16, 16)] = jnp.full((16,), val, ref.dtype)
            return 0
        lax.fori_loop(0, n // 16, body, 0, unroll=False)

    fill(zb, BINS // NS, 0.0)
    fill(ones, K, 1.0)

    pltpu.sync_copy(zb, binsS.at[pl.ds(sid * (BINS // NS), BINS // NS)])
    plsc.subcore_barrier()

    pltpu.sync_copy(dst_p.at[sid], dstA)
    pltpu.sync_copy(typ_p.at[sid], typA)
    pltpu.sync_copy(src_p.at[sid], srcA)

    def build_keys(j, kb):
        valid = (sid * NBT + j) < NB_REAL
        for q in range(K // 16):
            sl = pl.ds(q * 16, 16)
            kv = dstA[j, sl] * R + typA[j, sl]
            kb[sl] = jnp.where(valid, kv, DUMP_BIN)

    def hist_pair(t, _):
        j0 = 2 * t
        build_keys(j0, keyb)
        pltpu.async_copy(ones, binsS.at[keyb], hsemA, add=True)
        build_keys(j0 + 1, keyb2)
        pltpu.async_copy(ones, binsS.at[keyb2], hsemB, add=True)
        pltpu.make_async_copy(ones, binsS.at[keyb], hsemA).wait()
        pltpu.make_async_copy(ones, binsS.at[keyb2], hsemB).wait()
        return 0
    lax.fori_loop(0, NBT // 2, hist_pair, 0, unroll=False)
    plsc.subcore_barrier()

    def build_phase2(j, kb):
        jj = cid * NBH + j
        for q in range(K // 16):
            sl = pl.ds(q * 16, 16)
            dd = dstA[jj, sl]
            tt = typA[jj, sl]
            kb[sl] = dd * R + tt
            riB[j, sl] = ((tt * N + srcA[jj, sl]) << 14) | dd

    def emit_scale(j, db):
        jj = cid * NBH + j
        valid = (sid * NBT + jj) < NB_REAL
        for q in range(K // 16):
            sl = pl.ds(q * 16, 16)
            sv = 1.0 / jnp.maximum(db[sl], 1.0)
            sB[j, sl] = jnp.where(valid, sv, 0.0)

    def scale_pair(t, _):
        j0 = 2 * t
        build_phase2(j0, keyb)
        pltpu.async_copy(binsS.at[keyb], degb, hsemA)
        build_phase2(j0 + 1, keyb2)
        pltpu.async_copy(binsS.at[keyb2], degb2, hsemB)
        pltpu.make_async_copy(binsS.at[keyb], degb, hsemA).wait()
        emit_scale(j0, degb)
        pltpu.make_async_copy(binsS.at[keyb2], degb2, hsemB).wait()
        emit_scale(j0 + 1, degb2)
        return 0
    lax.fori_loop(0, NBH // 2, scale_pair, 0, unroll=False)

    pltpu.sync_copy(riB, combo_out.at[cid, sid])
    pltpu.sync_copy(sB, s_out.at[cid, sid])


CH = 40
_C0 = 3


@functools.cache
def _get_sc_aggregate():
    return functools.partial(
        pl.kernel,
        out_type=jax.ShapeDtypeStruct((NC, N_PAD, D), jnp.float32),
        mesh=_get_mesh(),
        scratch_types=[
            pltpu.VMEM((CH, K), jnp.int32),
            pltpu.VMEM((CH, K), jnp.int32),
            pltpu.VMEM((CH, K), jnp.float32),
            pltpu.VMEM((K, D), jnp.float32),
            pltpu.VMEM((K, D), jnp.float32),
            pltpu.SemaphoreType.DMA,
            pltpu.SemaphoreType.DMA,
            pltpu.VMEM_SHARED((N_PAD, D), jnp.float32),
        ],
    )(_sc_aggregate_body)


def _sc_aggregate_body(hw, combo4, s4, agg_out,
                       idxS, dstS, sS, rowsA, rowsB, semA, semB, accS):
    cid = lax.axis_index("c")
    sid = lax.axis_index("s")

    def zrow(i, _):
        for d in range(D // 16):
            rowsA[i, pl.ds(d * 16, 16)] = jnp.zeros((16,), jnp.float32)
        return 0
    lax.fori_loop(0, K, zrow, 0, unroll=False)
    base = sid * ROWS_PER_TILE
    for t in range(ROWS_PER_TILE // K):
        pltpu.sync_copy(rowsA, accS.at[pl.ds(base + t * K, K)])
    plsc.subcore_barrier()

    def scale_scatter(buf, j):
        def qstep(q, _):
            sv16 = sS[j, pl.ds(q * 16, 16)]
            for l in range(16):
                i = q * 16 + l
                sv = sv16[l]
                for d in range(D // 16):
                    sl = pl.ds(d * 16, 16)
                    buf[i, sl] = buf[i, sl] * sv
            return 0
        lax.fori_loop(0, K // 16, qstep, 0, unroll=False)
        pltpu.sync_copy(buf, accS.at[dstS.at[j]], add=True)

    def chunk(c, _):
        ph = NBH // CH
        hc = c // ph
        off = pl.multiple_of((c % ph) * CH, 8)
        pltpu.sync_copy(combo4.at[hc, sid, pl.ds(off, CH)], idxS)
        pltpu.sync_copy(s4.at[hc, sid, pl.ds(off, CH)], sS)

        def unpack(j, _):
            for q in range(K // 16):
                sl = pl.ds(q * 16, 16)
                cc = idxS[j, sl]
                dstS[j, sl] = cc & 16383
                idxS[j, sl] = cc >> 14
            return 0
        lax.fori_loop(0, CH, unpack, 0, unroll=False)

        pltpu.async_copy(hw.at[idxS.at[0]], rowsA, semA)

        def pair(t, _):
            jA = 2 * t
            jB = 2 * t + 1
            pltpu.async_copy(hw.at[idxS.at[jB]], rowsB, semB)
            pltpu.make_async_copy(hw.at[idxS.at[jA]], rowsA, semA).wait()
            scale_scatter(rowsA, jA)

            @pl.when(t < CH // 2 - 1)
            def _():
                pltpu.async_copy(hw.at[idxS.at[jA + 2]], rowsA, semA)

            pltpu.make_async_copy(hw.at[idxS.at[jB]], rowsB, semB).wait()
            scale_scatter(rowsB, jB)
            return 0
        lax.fori_loop(0, CH // 2, pair, 0, unroll=False)
        return 0

    nchunks = NBH * NC // CH
    lo = jnp.where(cid == 0, 0, _C0)
    hi = jnp.where(cid == 0, _C0, nchunks)
    lax.fori_loop(lo, hi, chunk, 0, unroll=False)
    plsc.subcore_barrier()

    pltpu.sync_copy(accS.at[pl.ds(base, ROWS_PER_TILE)],
                    agg_out.at[cid, pl.ds(base, ROWS_PER_TILE)])


_BLK = 400
_NBLK = N // _BLK


def _tc_proj_body(x_ref, wp_ref, bp_ref, w_ref, out_ref):
    h = jnp.dot(x_ref[...], wp_ref[...],
                preferred_element_type=jnp.float32) + bp_ref[...]
    for r in range(R):
        out_ref[r] = jnp.dot(h, w_ref[r], preferred_element_type=jnp.float32)


def _tc_proj(x, wp, bp, w):
    return pl.pallas_call(
        _tc_proj_body,
        grid=(_NBLK,),
        in_specs=[
            pl.BlockSpec((_BLK, D), lambda i: (i, 0)),
            pl.BlockSpec((D, D), lambda i: (0, 0)),
            pl.BlockSpec((1, D), lambda i: (0, 0)),
            pl.BlockSpec((R, D, D), lambda i: (0, 0, 0)),
        ],
        out_specs=pl.BlockSpec((R, _BLK, D), lambda i: (0, i, 0)),
        out_shape=jax.ShapeDtypeStruct((R, N, D), jnp.float32),
    )(x, wp, bp, w)


def _tc_comb_body(agg_ref, b_ref, w_ref, out_ref):
    h = jax.nn.relu(agg_ref[0] + agg_ref[1] + b_ref[...])
    for r in range(R):
        out_ref[r] = jnp.dot(h, w_ref[r], preferred_element_type=jnp.float32)


def _tc_comb(agg, b, w):
    return pl.pallas_call(
        _tc_comb_body,
        grid=(_NBLK,),
        in_specs=[
            pl.BlockSpec((NC, _BLK, D), lambda i: (0, i, 0)),
            pl.BlockSpec((1, D), lambda i: (0, 0)),
            pl.BlockSpec((R, D, D), lambda i: (0, 0, 0)),
        ],
        out_specs=pl.BlockSpec((R, _BLK, D), lambda i: (0, i, 0)),
        out_shape=jax.ShapeDtypeStruct((R, N, D), jnp.float32),
    )(agg, b, w)


def _tc_final_body(agg_ref, b_ref, out_ref):
    out_ref[...] = jax.nn.relu(agg_ref[0] + agg_ref[1] + b_ref[...])


def _tc_final(agg, b):
    return pl.pallas_call(
        _tc_final_body,
        grid=(_NBLK,),
        in_specs=[
            pl.BlockSpec((NC, _BLK, D), lambda i: (0, i, 0)),
            pl.BlockSpec((1, D), lambda i: (0, 0)),
        ],
        out_specs=pl.BlockSpec((_BLK, D), lambda i: (i, 0)),
        out_shape=jax.ShapeDtypeStruct((N, D), jnp.float32),
    )(agg, b)


def kernel(x, edge_index, edge_type, W_proj, b_proj, W1, b1, W2, b2):
    pad = E_PAD - E
    src = jnp.concatenate([edge_index[0], jnp.zeros((pad,), jnp.int32)])
    dst = jnp.concatenate([edge_index[1], jnp.zeros((pad,), jnp.int32)])
    et = jnp.concatenate([edge_type, jnp.zeros((pad,), jnp.int32)])

    dst_p = dst.reshape(NS, NBT, K)
    typ_p = et.reshape(NS, NBT, K)
    src_p = src.reshape(NS, NBT, K)

    sc_degscale = _get_sc_degscale()
    sc_aggregate = _get_sc_aggregate()
    combo4, s4 = sc_degscale(dst_p, typ_p, src_p)

    hw1 = _tc_proj(x, W_proj, b_proj.reshape(1, D), W1)
    agg1 = sc_aggregate(hw1.reshape(R * N, D), combo4, s4)
    hw2 = _tc_comb(agg1, b1.reshape(1, D), W2)
    agg2 = sc_aggregate(hw2.reshape(R * N, D), combo4, s4)
    return _tc_final(agg2, b2.reshape(1, D))
